# Initial kernel scaffold; baseline (speedup 1.0000x reference)
import jax
import jax.numpy as jnp
from jax.experimental import pallas as pl


def kernel(word, table):
    raise NotImplementedError("write your pallas kernel here")


# speedup vs baseline: 1.0233x; 1.0233x over previous
import functools

import jax
import jax.numpy as jnp
from jax import lax
from jax.experimental import pallas as pl
from jax.experimental.pallas import tpu as pltpu
from jax.experimental.pallas import tpu_sc as plsc

VOCAB_SIZE = 1000000
EMBED_DIM = 32
BATCH = 16384
HIST = 50

_info = plsc.get_sparse_core_info()
NC = _info.num_cores
NS = _info.num_subcores
NW = NC * NS

B = BATCH * HIST
B_PER_W = B // NW
CHUNK = 128
N_CHUNKS = B_PER_W // CHUNK


def _embed_body(idx_hbm, table_hbm, out_hbm, idx_v, rows_v, gsem):
    wid = lax.axis_index("s") * NC + lax.axis_index("c")
    base = wid * B_PER_W

    pltpu.sync_copy(idx_hbm.at[wid], idx_v)

    def chunk_step(j, carry):
        pltpu.async_copy(table_hbm.at[idx_v.at[j]], rows_v, gsem).wait()
        pltpu.sync_copy(rows_v, out_hbm.at[pl.ds(base + j * CHUNK, CHUNK)])
        return carry

    lax.fori_loop(0, N_CHUNKS, chunk_step, 0, unroll=False)


@jax.jit
def _embed(word_flat, table):
    mesh = plsc.VectorSubcoreMesh(core_axis_name="c", subcore_axis_name="s")
    k = pl.kernel(
        _embed_body,
        out_type=jax.ShapeDtypeStruct((B, EMBED_DIM), jnp.float32),
        mesh=mesh,
        scratch_types=[
            pltpu.VMEM((N_CHUNKS, CHUNK), jnp.int32),
            pltpu.VMEM((CHUNK, EMBED_DIM), jnp.float32),
            pltpu.SemaphoreType.DMA,
        ],
        compiler_params=pltpu.CompilerParams(use_tc_tiling_on_sc=False),
    )
    return k(word_flat, table)


def kernel(word, table):
    word_flat = word.reshape(NW, N_CHUNKS, CHUNK).astype(jnp.int32)
    out = _embed(word_flat, table)
    return out.reshape(BATCH, HIST, EMBED_DIM)


# speedup vs baseline: 1.1137x; 1.0884x over previous
import jax
import jax.numpy as jnp
from jax import lax
from jax.experimental import pallas as pl
from jax.experimental.pallas import tpu as pltpu
from jax.experimental.pallas import tpu_sc as plsc

VOCAB_SIZE = 1000000
EMBED_DIM = 32
BATCH = 16384
HIST = 50

_info = plsc.get_sparse_core_info()
NC = _info.num_cores
NS = _info.num_subcores
NW = NC * NS

B = BATCH * HIST
B_PER_W = B // NW
CHUNK = 128
N_CHUNKS = B_PER_W // CHUNK
G_CHUNKS = 5
G_ROWS = G_CHUNKS * CHUNK
NG = N_CHUNKS // G_CHUNKS
NBUF = 4
NBLK = NG // NBUF


def _embed_body(idx_hbm, table_hbm, out_hbm, idx_v, rows_v, gsems, wsems):
    wid = lax.axis_index("s") * NC + lax.axis_index("c")
    base = wid * B_PER_W

    pltpu.sync_copy(idx_hbm.at[wid], idx_v)

    def issue_gathers(n, bn):
        for c in range(G_CHUNKS):
            pltpu.async_copy(
                table_hbm.at[idx_v.at[n * G_CHUNKS + c]],
                rows_v.at[bn, pl.ds(c * CHUNK, CHUNK)],
                gsems[bn],
            )

    def drain_gathers(b):
        pltpu.make_async_copy(
            rows_v.at[b], out_hbm.at[pl.ds(0, G_ROWS)], gsems[b]
        ).wait()

    def issue_writeout(s, b):
        pltpu.async_copy(
            rows_v.at[b], out_hbm.at[pl.ds(base + s * G_ROWS, G_ROWS)], wsems[b]
        )

    def drain_writeout(b):
        pltpu.make_async_copy(
            rows_v.at[b], out_hbm.at[pl.ds(0, G_ROWS)], wsems[b]
        ).wait()

    def body(s, b, ahead_valid, ahead_wait):
        if ahead_valid:
            bn = (b + 2) % NBUF
            if ahead_wait:
                drain_writeout(bn)
            issue_gathers(s + 2, bn)
        drain_gathers(b)
        issue_writeout(s, b)

    issue_gathers(0, 0)
    issue_gathers(1, 1)
    for b in range(NBUF):
        body(b, b, ahead_valid=True, ahead_wait=(b + 2 >= NBUF))

    def block_step(t, carry):
        s0 = t * NBUF
        for b in range(NBUF):
            body(s0 + b, b, ahead_valid=True, ahead_wait=True)
        return carry

    lax.fori_loop(1, NBLK - 1, block_step, 0, unroll=False)

    s0 = (NBLK - 1) * NBUF
    for b in range(NBUF):
        body(s0 + b, b, ahead_valid=(s0 + b + 2 < NG), ahead_wait=True)
    for b in range(NBUF):
        drain_writeout(b)


@jax.jit
def _embed(word_flat, table):
    mesh = plsc.VectorSubcoreMesh(core_axis_name="c", subcore_axis_name="s")
    k = pl.kernel(
        _embed_body,
        out_type=jax.ShapeDtypeStruct((B, EMBED_DIM), jnp.float32),
        mesh=mesh,
        scratch_types=[
            pltpu.VMEM((N_CHUNKS, CHUNK), jnp.int32),
            pltpu.VMEM((NBUF, G_ROWS, EMBED_DIM), jnp.float32),
            [pltpu.SemaphoreType.DMA] * NBUF,
            [pltpu.SemaphoreType.DMA] * NBUF,
        ],
        compiler_params=pltpu.CompilerParams(use_tc_tiling_on_sc=False),
    )
    return k(word_flat, table)


def kernel(word, table):
    word_flat = word.reshape(NW, N_CHUNKS, CHUNK).astype(jnp.int32)
    out = _embed(word_flat, table)
    return out.reshape(BATCH, HIST, EMBED_DIM)


# speedup vs baseline: 1.9388x; 1.7409x over previous
import jax
import jax.numpy as jnp
from jax import lax
from jax.experimental import pallas as pl
from jax.experimental.pallas import tpu as pltpu
from jax.experimental.pallas import tpu_sc as plsc

VOCAB_SIZE = 1000000
EMBED_DIM = 32
BATCH = 16384
HIST = 50

_info = plsc.get_sparse_core_info()
NC = _info.num_cores
NS = _info.num_subcores
NW = NC * NS

B_PER_W = BATCH // NW
CHUNK = 128
N_CHUNKS = B_PER_W // CHUNK


def _embed_body(idx_hbm, table_hbm, out_hbm, idx_v, rows_v, gsems, wsems):
    wid = lax.axis_index("s") * NC + lax.axis_index("c")
    base = wid * B_PER_W

    for c in range(N_CHUNKS):
        pltpu.sync_copy(
            idx_hbm.at[:, pl.ds(base + c * CHUNK, CHUNK)], idx_v.at[c]
        )

    def issue_gathers(h, b):
        for c in range(N_CHUNKS):
            pltpu.async_copy(
                table_hbm.at[idx_v.at[c, h]],
                rows_v.at[b, pl.ds(c * CHUNK, CHUNK)],
                gsems[b],
            )

    def drain_gathers(b):
        pltpu.make_async_copy(
            table_hbm.at[pl.ds(0, B_PER_W)], rows_v.at[b], gsems[b]
        ).wait()

    def issue_writeout(h, b):
        pltpu.async_copy(
            rows_v.at[b], out_hbm.at[h, pl.ds(base, B_PER_W)], wsems[b]
        )

    def drain_writeout(b):
        pltpu.make_async_copy(
            table_hbm.at[pl.ds(0, B_PER_W)], rows_v.at[b], wsems[b]
        ).wait()

    def body(h, bh, b, ahead, first):
        if ahead:
            if not first:
                drain_writeout(b)
            issue_gathers(h + 2, b)
        drain_gathers(bh)
        issue_writeout(h, bh)

    issue_gathers(0, 0)
    issue_gathers(1, 1)
    body(0, 0, 2, ahead=True, first=True)
    body(1, 1, 3, ahead=True, first=True)

    def quad_step(t, carry):
        for b in range(4):
            h = 4 * t + 2 + b
            body(h, (2 + b) % 4, b, ahead=True, first=False)
        return carry

    lax.fori_loop(0, (HIST - 6) // 4, quad_step, 0, unroll=False)

    for b in range(4):
        h = HIST - 4 + b
        body(h, (2 + b) % 4, b, ahead=(h + 2 < HIST), first=False)
    for h in range(HIST - 4, HIST):
        drain_writeout((2 + h - (HIST - 4)) % 4)


@jax.jit
def _embed(word_t, table):
    mesh = plsc.VectorSubcoreMesh(core_axis_name="c", subcore_axis_name="s")
    k = pl.kernel(
        _embed_body,
        out_type=jax.ShapeDtypeStruct((HIST, BATCH, EMBED_DIM), jnp.float32),
        mesh=mesh,
        scratch_types=[
            pltpu.VMEM((N_CHUNKS, HIST, CHUNK), jnp.int32),
            pltpu.VMEM((4, B_PER_W, EMBED_DIM), jnp.float32),
            [pltpu.SemaphoreType.DMA] * 4,
            [pltpu.SemaphoreType.DMA] * 4,
        ],
        compiler_params=pltpu.CompilerParams(use_tc_tiling_on_sc=False),
    )
    return k(word_t, table)


def kernel(word, table):
    word_t = word.T.astype(jnp.int32)
    out = _embed(word_t, table)
    return jnp.transpose(out, (1, 0, 2))
